# Initial kernel scaffold; baseline (speedup 1.0000x reference)
import jax
import jax.numpy as jnp
from jax.experimental import pallas as pl


def kernel(x, edge_index, W1, b1, W2, b2):
    raise NotImplementedError("write your pallas kernel here")


# speedup vs baseline: 30.0389x; 30.0389x over previous
import functools

import jax
import jax.numpy as jnp
from jax import lax
from jax.experimental import pallas as pl
from jax.experimental.pallas import tpu as pltpu
from jax.experimental.pallas import tpu_sc as plsc

N = 10000
E = 320000
D_IN = 128
DH = 16
NCLS = 40

NC = 2
NS = 16
NW = NC * NS
EPT = E // NW
CHUNK = 80
NCHUNK = EPT // CHUNK
NPAD = 10240
RPT = NPAD // NS

_mesh = plsc.VectorSubcoreMesh(core_axis_name="c", subcore_axis_name="s")


@functools.partial(
    pl.kernel,
    out_type=jax.ShapeDtypeStruct((NC, NS, N), jnp.float32),
    mesh=_mesh,
    compiler_params=pltpu.CompilerParams(needs_layout_passes=False),
    scratch_types=[
        pltpu.VMEM((EPT,), jnp.int32),
        pltpu.VMEM((N,), jnp.float32),
    ],
)
def _deg_kernel(dst_hbm, degp_hbm, dstv, degv):
    c = lax.axis_index("c")
    s = lax.axis_index("s")
    pltpu.sync_copy(dst_hbm.at[c, s], dstv)

    zeros16 = jnp.zeros((16,), jnp.float32)

    def zbody(i, _):
        degv[pl.ds(i * 16, 16)] = zeros16
        return 0

    lax.fori_loop(0, N // 16, zbody, 0)

    ones16 = jnp.ones((16,), jnp.float32)

    def body(i, _):
        v = dstv[pl.ds(i * 16, 16)]
        plsc.addupdate_scatter(degv, [v], ones16)
        return 0

    lax.fori_loop(0, EPT // 16, body, 0)
    pltpu.sync_copy(degv, degp_hbm.at[c, s])


@functools.partial(
    pl.kernel,
    out_type=jax.ShapeDtypeStruct((NC, NPAD, DH), jnp.float32),
    mesh=_mesh,
    compiler_params=pltpu.CompilerParams(
        needs_layout_passes=False, use_tc_tiling_on_sc=False),
    scratch_types=[
        pltpu.VMEM((NCHUNK, CHUNK), jnp.int32),
        pltpu.VMEM((NCHUNK, CHUNK), jnp.int32),
        pltpu.VMEM((CHUNK, DH), jnp.float32),
        pltpu.VMEM((RPT, DH), jnp.float32),
        pltpu.VMEM_SHARED((NPAD, DH), jnp.float32),
        pltpu.SemaphoreType.DMA,
    ],
)
def _agg_kernel(g_hbm, src_hbm, dst_hbm, accp_hbm, srcv, dstv, gbuf, obuf,
                acc, sem):
    c = lax.axis_index("c")
    s = lax.axis_index("s")
    pltpu.sync_copy(src_hbm.at[c, s], srcv)
    pltpu.sync_copy(dst_hbm.at[c, s], dstv)

    zeros16 = jnp.zeros((16,), jnp.float32)

    def zbody(i, _):
        obuf[i] = zeros16
        return 0

    lax.fori_loop(0, RPT, zbody, 0)
    pltpu.sync_copy(obuf, acc.at[pl.ds(s * RPT, RPT)])
    plsc.subcore_barrier()

    def body(j, _):
        pltpu.async_copy(g_hbm.at[srcv.at[j]], gbuf, sem).wait()
        pltpu.sync_copy(gbuf, acc.at[dstv.at[j]], add=True)
        return 0

    lax.fori_loop(0, NCHUNK, body, 0)
    plsc.subcore_barrier()
    pltpu.sync_copy(acc.at[pl.ds(s * RPT, RPT)], obuf)
    pltpu.sync_copy(obuf, accp_hbm.at[c].at[pl.ds(s * RPT, RPT)])


def _prep1_body(x_ref, w1_ref, degp_ref, g_ref, dis_ref):
    deg = jnp.sum(degp_ref[...], axis=1, keepdims=True) + 1.0
    dis = lax.rsqrt(deg)
    h = jnp.dot(x_ref[...], w1_ref[...], preferred_element_type=jnp.float32)
    g_ref[...] = h * dis
    dis_ref[...] = dis


_prep1 = pl.pallas_call(
    _prep1_body,
    out_shape=(
        jax.ShapeDtypeStruct((N, DH), jnp.float32),
        jax.ShapeDtypeStruct((N, 1), jnp.float32),
    ),
)


def _prep2_body(accp_ref, g_ref, dis_ref, b1_ref, g2_ref):
    dis = dis_ref[...]
    agg = (accp_ref[0] + accp_ref[1])[:N] + g_ref[...]
    h1 = jnp.maximum(agg * dis + b1_ref[...], 0.0)
    g2_ref[...] = h1 * dis


_prep2 = pl.pallas_call(
    _prep2_body,
    out_shape=jax.ShapeDtypeStruct((N, DH), jnp.float32),
)


def _final_body(accp_ref, g2_ref, dis_ref, w2_ref, b2_ref, out_ref):
    t = ((accp_ref[0] + accp_ref[1])[:N] + g2_ref[...]) * dis_ref[...]
    z = jnp.dot(t, w2_ref[...], preferred_element_type=jnp.float32)
    z = z + b2_ref[...]
    m = jnp.max(z, axis=1, keepdims=True)
    lse = m + jnp.log(jnp.sum(jnp.exp(z - m), axis=1, keepdims=True))
    out_ref[...] = z - lse


_final = pl.pallas_call(
    _final_body,
    out_shape=jax.ShapeDtypeStruct((N, NCLS), jnp.float32),
)


def kernel(x, edge_index, W1, b1, W2, b2):
    src = edge_index[0].reshape(NC, NS, NCHUNK, CHUNK)
    dst = edge_index[1].reshape(NC, NS, NCHUNK, CHUNK)
    dst_flat = edge_index[1].reshape(NC, NS, EPT)

    degp = _deg_kernel(dst_flat)
    degp_t = degp.reshape(NW, N).T

    g, dis = _prep1(x, W1, degp_t)
    accp1 = _agg_kernel(g, src, dst)
    g2 = _prep2(accp1, g, dis, b1.reshape(1, DH))
    accp2 = _agg_kernel(g2, src, dst)
    return _final(accp2, g2, dis, W2, b2.reshape(1, NCLS))


# speedup vs baseline: 42.7591x; 1.4235x over previous
import functools

import jax
import jax.numpy as jnp
from jax import lax
from jax.experimental import pallas as pl
from jax.experimental.pallas import tpu as pltpu
from jax.experimental.pallas import tpu_sc as plsc

N = 10000
E = 320000
D_IN = 128
DH = 16
NCLS = 40

NC = 2
NS = 16
NW = NC * NS
EPT = E // NW
CHUNK = 80
NCHUNK = EPT // CHUNK
NPAD = 10240
RPT = NPAD // NS

_mesh = plsc.VectorSubcoreMesh(core_axis_name="c", subcore_axis_name="s")


@functools.partial(
    pl.kernel,
    out_type=jax.ShapeDtypeStruct((NC, NS, N), jnp.float32),
    mesh=_mesh,
    compiler_params=pltpu.CompilerParams(needs_layout_passes=False),
    scratch_types=[
        pltpu.VMEM((EPT,), jnp.int32),
        pltpu.VMEM((N,), jnp.float32),
    ],
)
def _deg_kernel(dst_hbm, degp_hbm, dstv, degv):
    c = lax.axis_index("c")
    s = lax.axis_index("s")
    pltpu.sync_copy(dst_hbm.at[c, s], dstv)

    zeros16 = jnp.zeros((16,), jnp.float32)

    def zbody(i, _):
        degv[pl.ds(i * 16, 16)] = zeros16
        return 0

    lax.fori_loop(0, N // 16, zbody, 0)

    ones16 = jnp.ones((16,), jnp.float32)

    def body(i, _):
        v = dstv[pl.ds(i * 16, 16)]
        plsc.addupdate_scatter(degv, [v], ones16)
        return 0

    lax.fori_loop(0, EPT // 16, body, 0)
    pltpu.sync_copy(degv, degp_hbm.at[c, s])


@functools.partial(
    pl.kernel,
    out_type=jax.ShapeDtypeStruct((NC, NPAD, DH), jnp.float32),
    mesh=_mesh,
    compiler_params=pltpu.CompilerParams(
        needs_layout_passes=False, use_tc_tiling_on_sc=False),
    scratch_types=[
        pltpu.VMEM((NCHUNK, CHUNK), jnp.int32),
        pltpu.VMEM((NCHUNK, CHUNK), jnp.int32),
        pltpu.VMEM((CHUNK, DH), jnp.float32),
        pltpu.VMEM((CHUNK, DH), jnp.float32),
        pltpu.VMEM((RPT, DH), jnp.float32),
        pltpu.VMEM_SHARED((NPAD, DH), jnp.float32),
        pltpu.SemaphoreType.DMA,
        pltpu.SemaphoreType.DMA,
    ],
)
def _agg_kernel(g_hbm, src_hbm, dst_hbm, accp_hbm, srcv, dstv, gbuf0, gbuf1,
                obuf, acc, sem0, sem1):
    c = lax.axis_index("c")
    s = lax.axis_index("s")
    pltpu.sync_copy(src_hbm.at[c, s], srcv)
    pltpu.sync_copy(dst_hbm.at[c, s], dstv)

    zeros16 = jnp.zeros((16,), jnp.float32)

    def zbody(i, _):
        obuf[i] = zeros16
        return 0

    lax.fori_loop(0, RPT, zbody, 0)
    pltpu.sync_copy(obuf, acc.at[pl.ds(s * RPT, RPT)])
    plsc.subcore_barrier()

    def start(j, buf, sem):
        pltpu.async_copy(g_hbm.at[srcv.at[j]], buf, sem)

    def finish(j, buf, sem):
        pltpu.make_async_copy(g_hbm.at[srcv.at[j]], buf, sem).wait()
        pltpu.sync_copy(buf, acc.at[dstv.at[j]], add=True)

    start(0, gbuf0, sem0)

    def body(k, _):
        j0 = 2 * k
        start(j0 + 1, gbuf1, sem1)
        finish(j0, gbuf0, sem0)
        start(j0 + 2, gbuf0, sem0)
        finish(j0 + 1, gbuf1, sem1)
        return 0

    lax.fori_loop(0, (NCHUNK - 1) // 2, body, 0)
    finish(NCHUNK - 1, gbuf0, sem0)
    plsc.subcore_barrier()
    pltpu.sync_copy(acc.at[pl.ds(s * RPT, RPT)], obuf)
    pltpu.sync_copy(obuf, accp_hbm.at[c].at[pl.ds(s * RPT, RPT)])


def _prep1_body(x_ref, w1_ref, degp_ref, g_ref, dis_ref):
    deg = jnp.sum(degp_ref[...], axis=1, keepdims=True) + 1.0
    dis = lax.rsqrt(deg)
    h = jnp.dot(x_ref[...], w1_ref[...], preferred_element_type=jnp.float32)
    g_ref[...] = h * dis
    dis_ref[...] = dis


_prep1 = pl.pallas_call(
    _prep1_body,
    out_shape=(
        jax.ShapeDtypeStruct((N, DH), jnp.float32),
        jax.ShapeDtypeStruct((N, 1), jnp.float32),
    ),
)


def _prep2_body(accp_ref, g_ref, dis_ref, b1_ref, g2_ref):
    dis = dis_ref[...]
    agg = (accp_ref[0] + accp_ref[1])[:N] + g_ref[...]
    h1 = jnp.maximum(agg * dis + b1_ref[...], 0.0)
    g2_ref[...] = h1 * dis


_prep2 = pl.pallas_call(
    _prep2_body,
    out_shape=jax.ShapeDtypeStruct((N, DH), jnp.float32),
)


def _final_body(accp_ref, g2_ref, dis_ref, w2_ref, b2_ref, out_ref):
    t = ((accp_ref[0] + accp_ref[1])[:N] + g2_ref[...]) * dis_ref[...]
    z = jnp.dot(t, w2_ref[...], preferred_element_type=jnp.float32)
    z = z + b2_ref[...]
    m = jnp.max(z, axis=1, keepdims=True)
    lse = m + jnp.log(jnp.sum(jnp.exp(z - m), axis=1, keepdims=True))
    out_ref[...] = z - lse


_final = pl.pallas_call(
    _final_body,
    out_shape=jax.ShapeDtypeStruct((N, NCLS), jnp.float32),
)


def kernel(x, edge_index, W1, b1, W2, b2):
    src = edge_index[0].reshape(NC, NS, NCHUNK, CHUNK)
    dst = edge_index[1].reshape(NC, NS, NCHUNK, CHUNK)
    dst_flat = edge_index[1].reshape(NC, NS, EPT)

    degp = _deg_kernel(dst_flat)
    degp_t = degp.reshape(NW, N).T

    g, dis = _prep1(x, W1, degp_t)
    accp1 = _agg_kernel(g, src, dst)
    g2 = _prep2(accp1, g, dis, b1.reshape(1, DH))
    accp2 = _agg_kernel(g2, src, dst)
    return _final(accp2, g2, dis, W2, b2.reshape(1, NCLS))
